# baseline (speedup 1.0000x reference)
import functools

import jax
import jax.numpy as jnp
from jax import lax
from jax.experimental import pallas as pl
from jax.experimental.pallas import tpu as pltpu
from jax.experimental.pallas import tpu_sc as plsc

V = 50000
K = 512
D = 256
N = 32768

NC = 2
NS = 16
NW = NC * NS
TOK_PER_TILE = N // NW
GCHUNK = 128
NCHUNK = TOK_PER_TILE // GCHUNK


def _sc_gather_bincount(doc, emb_w, zeros_v, ones_g):
    mesh = plsc.VectorSubcoreMesh(core_axis_name="c", subcore_axis_name="s")

    @functools.partial(
        pl.kernel,
        mesh=mesh,
        out_type=(
            jax.ShapeDtypeStruct((N, D), jnp.float32),
            jax.ShapeDtypeStruct((NC, V), jnp.float32),
        ),
        scratch_types=[
            pltpu.VMEM((NCHUNK, GCHUNK), jnp.int32),
            pltpu.VMEM((3, GCHUNK, D), jnp.float32),
            pltpu.VMEM((GCHUNK,), jnp.float32),
            pltpu.VMEM_SHARED((V,), jnp.float32),
            pltpu.SemaphoreType.DMA((3,)),
            pltpu.SemaphoreType.DMA((3,)),
        ],
    )
    def k(doc_hbm, emb_hbm, zeros_hbm, ones_hbm, out_hbm, bc_hbm,
          idx_v, rows_v, ones_v, hist_sh, gsem, wsem):
        cid = lax.axis_index("c")
        sid = lax.axis_index("s")
        wid = sid * NC + cid
        base = wid * TOK_PER_TILE

        @pl.when(sid == 0)
        def _():
            pltpu.sync_copy(zeros_hbm, hist_sh)

        pltpu.sync_copy(ones_hbm, ones_v)
        pltpu.sync_copy(doc_hbm.at[wid], idx_v)
        plsc.subcore_barrier()

        def gather(j):
            return pltpu.async_copy(emb_hbm.at[idx_v.at[j]],
                                    rows_v.at[j % 3], gsem.at[j % 3])

        def write(j):
            return pltpu.async_copy(rows_v.at[j % 3],
                                    out_hbm.at[pl.ds(base + j * GCHUNK,
                                                     GCHUNK)],
                                    wsem.at[j % 3])

        gathers = [None] * NCHUNK
        writes = [None] * NCHUNK
        gathers[0] = gather(0)
        gathers[1] = gather(1)
        for j in range(NCHUNK):
            if j + 2 < NCHUNK:
                if j >= 1:
                    writes[j - 1].wait()
                gathers[j + 2] = gather(j + 2)
            gathers[j].wait()
            writes[j] = write(j)
            pltpu.sync_copy(ones_v, hist_sh.at[idx_v.at[j]], add=True)
        for j in range(max(0, NCHUNK - 3), NCHUNK):
            writes[j].wait()

        plsc.subcore_barrier()

        @pl.when(sid == 0)
        def _():
            pltpu.sync_copy(hist_sh, bc_hbm.at[cid])

    return k(doc, emb_w, zeros_v, ones_g)


BN = 4096
NB = N // BN


BV = 4096
NVB = (V + BV - 1) // BV
VPAD = NVB * BV


def _mega_body(e_ref, c_ref, w_ref, b_ref, bc_ref,
               enc_ref, qw_ref, docu_ref, out_ref, vq_ref, lts_ref,
               acc_ref, vqs_ref, mm_ref, ss_ref, docu_v, lg_scr):
    t = pl.program_id(0)

    @pl.when(t < NB)
    def _():
        _vq_step(t, e_ref, c_ref, enc_ref, qw_ref, vq_ref, lts_ref,
                 acc_ref, vqs_ref)

    @pl.when(t == NB)
    def _():
        docu_v[...] = lax.dot_general(acc_ref[...], c_ref[...],
                                      (((1,), (0,)), ((), ()))) / N
        docu_ref[...] = docu_v[...]
        mm_ref[0, 0] = -jnp.inf
        ss_ref[0, 0] = 0.0

    @pl.when((t >= NB) & (t < NB + NVB))
    def _():
        j = t - NB
        w = w_ref[...]
        docu = docu_v[...]
        lg = lax.dot_general(docu, w, (((1,), (1,)), ((), ()))) + b_ref[...]
        lg_scr[0:1, pl.ds(j * BV, BV)] = lg
        viota = lax.broadcasted_iota(jnp.int32, (1, BV), 1) + j * BV
        valid = viota < V
        lgv = jnp.where(valid, lg, -jnp.inf)
        bm = jnp.max(lgv)
        m_old = mm_ref[0, 0]
        m_new = jnp.maximum(m_old, bm)
        ssum = jnp.sum(jnp.where(valid, jnp.exp(lg - m_new), 0.0))
        ss_ref[0, 0] = ss_ref[0, 0] * jnp.exp(m_old - m_new) + ssum
        mm_ref[0, 0] = m_new

    @pl.when(t >= NB + NVB)
    def _():
        j = t - NB - NVB
        lg = lg_scr[0:1, pl.ds(j * BV, BV)]
        smax = jnp.exp(lg - mm_ref[0, 0]) / ss_ref[0, 0]
        bc = jnp.sum(bc_ref[...], axis=0, keepdims=True)
        out_ref[...] = jnp.log(smax + 1e-6) * bc


def _vq_step(i, e_ref, c_ref, enc_ref, qw_ref, vq_ref, lts_ref,
             acc_ref, vqs_ref):
    e = e_ref[...]
    c = c_ref[...]

    @pl.when(i == 0)
    def _():
        g = lax.dot_general(c, c, (((1,), (1,)), ((), ())))
        nrm = jnp.sum(c * c, axis=1)
        sm = jnp.sum(c, axis=1)
        d2 = (nrm[:, None] + nrm[None, :] - 2.0 * g
              + 2e-6 * (sm[:, None] - sm[None, :]) + D * 1e-12)
        dist = jnp.sqrt(jnp.maximum(d2, 0.0))
        r = lax.broadcasted_iota(jnp.int32, (K, K), 0)
        cc = lax.broadcasted_iota(jnp.int32, (K, K), 1)
        losses = jnp.where(r == cc, dist, jnp.maximum(0.0, 1.0 - dist))
        lts_ref[0, 0] = jnp.sum(losses) / (K * K)
    e2 = jnp.sum(e * e, axis=1, keepdims=True)
    c2 = jnp.sum(c * c, axis=1)
    cross = lax.dot_general(e, c, (((1,), (1,)), ((), ())))
    dist = e2 + c2[None, :] - 2.0 * cross
    m = jnp.min(dist, axis=1, keepdims=True)
    kiota = lax.broadcasted_iota(jnp.int32, (BN, K), 1)
    idx = jnp.min(jnp.where(dist == m, kiota, K), axis=1, keepdims=True)
    onehot = (kiota == idx).astype(jnp.float32)
    enc_ref[...] = onehot
    qw_ref[...] = jnp.dot(onehot, c)

    @pl.when(i == 0)
    def _():
        acc_ref[...] = jnp.zeros_like(acc_ref)
        vqs_ref[0, 0] = 0.0

    acc_ref[...] += jnp.sum(onehot, axis=0, keepdims=True)
    vqs_ref[0, 0] += jnp.sum(m)

    @pl.when(i == NB - 1)
    def _():
        mloss = vqs_ref[0, 0] / (N * D)
        vq_ref[0, 0] = mloss + 0.25 * mloss


def _tc_mega(embedded, cw, q2v_W, q2v_b2d, bc2):
    nb1 = NB - 1

    return pl.pallas_call(
        _mega_body,
        grid=(NB + 2 * NVB,),
        in_specs=[
            pl.BlockSpec((BN, D), lambda t: (jnp.minimum(t, nb1), 0)),
            pl.BlockSpec((K, D), lambda t: (0, 0)),
            pl.BlockSpec((BV, D),
                         lambda t: (jnp.clip(t - NB, 0, NVB - 1), 0)),
            pl.BlockSpec((1, BV),
                         lambda t: (0, jnp.clip(t - NB, 0, NVB - 1))),
            pl.BlockSpec((NC, BV),
                         lambda t: (0, jnp.clip(t - NB - NVB, 0, NVB - 1))),
        ],
        out_specs=[
            pl.BlockSpec((BN, K), lambda t: (jnp.minimum(t, nb1), 0)),
            pl.BlockSpec((BN, D), lambda t: (jnp.minimum(t, nb1), 0)),
            pl.BlockSpec((1, D), lambda t: (0, 0)),
            pl.BlockSpec((1, BV),
                         lambda t: (0, jnp.clip(t - NB - NVB, 0, NVB - 1))),
            pl.BlockSpec((1, 1), lambda t: (0, 0), memory_space=pltpu.SMEM),
            pl.BlockSpec((1, 1), lambda t: (0, 0), memory_space=pltpu.SMEM),
        ],
        out_shape=[
            jax.ShapeDtypeStruct((N, K), jnp.float32),
            jax.ShapeDtypeStruct((N, D), jnp.float32),
            jax.ShapeDtypeStruct((1, D), jnp.float32),
            jax.ShapeDtypeStruct((1, V), jnp.float32),
            jax.ShapeDtypeStruct((1, 1), jnp.float32),
            jax.ShapeDtypeStruct((1, 1), jnp.float32),
        ],
        scratch_shapes=[
            pltpu.VMEM((1, K), jnp.float32),
            pltpu.SMEM((1, 1), jnp.float32),
            pltpu.SMEM((1, 1), jnp.float32),
            pltpu.SMEM((1, 1), jnp.float32),
            pltpu.VMEM((1, D), jnp.float32),
            pltpu.VMEM((1, VPAD), jnp.float32),
        ],
    )(embedded, cw, q2v_W, q2v_b2d, bc2)


def kernel(input_document, emb_w, emb_concept_w, q2v_W, q2v_b):
    doc = input_document.astype(jnp.int32)
    zeros_v = jnp.zeros((V,), jnp.float32)
    ones_g = jnp.ones((GCHUNK,), jnp.float32)
    embedded, bc2 = _sc_gather_bincount(doc.reshape(NW, NCHUNK, GCHUNK),
                                        emb_w, zeros_v, ones_g)
    enc, qw, docu, outs, vq, lts = _tc_mega(
        embedded, emb_concept_w, q2v_W, q2v_b.reshape(1, V), bc2)
    return (enc, qw, docu, outs, vq.reshape(()), lts.reshape(()))
